# baseline (speedup 1.0000x reference)
import jax
import jax.numpy as jnp
from jax.experimental import pallas as pl
from jax.experimental.pallas import tpu as pltpu

N = 20000
R = 1000
NB = N // R
EPS = 1e-5

_NT = (((1,), (1,)), ((), ()))


def _stats_kernel(bp_ref, posw_ref, posb_ref, g4_ref, be4_ref,
                  b5_ref, c_ref):
    x = bp_ref[...]
    r = jnp.roll(x, -2, axis=1)
    wh = r - x + 1.0
    ctr = x + 0.5 * wh
    rc = jnp.sum(ctr, axis=0, keepdims=True)
    rw = jnp.sum(wh, axis=0, keepdims=True)
    rc2 = jnp.sum(ctr * ctr, axis=0, keepdims=True)
    rw2 = jnp.sum(wh * wh, axis=0, keepdims=True)
    for k in (5, 10, 20, 40, 80):
        rc = rc + jnp.roll(rc, -k, axis=1)
        rw = rw + jnp.roll(rw, -k, axis=1)
        rc2 = rc2 + jnp.roll(rc2, -k, axis=1)
        rw2 = rw2 + jnp.roll(rw2, -k, axis=1)
    mu4 = jnp.concatenate([rc[:, 1:3], rw[:, 1:3]], axis=1) / N
    ex2 = jnp.concatenate([rc2[:, 1:3], rw2[:, 1:3]], axis=1) / N
    var4 = ex2 - mu4 * mu4
    scale4 = g4_ref[...] * jax.lax.rsqrt(var4 + EPS)
    shift4 = be4_ref[...] - mu4 * scale4
    a4t = (posw_ref[...] * scale4).T
    beff = posb_ref[...] + jnp.sum(posw_ref[...] * shift4, axis=1)[None, :]
    zrow = jnp.zeros((1, 128), jnp.float32)
    b5_ref[...] = jnp.concatenate([
        zrow,
        0.5 * a4t[0:1] - a4t[2:3],
        0.5 * a4t[1:2] - a4t[3:4],
        0.5 * a4t[0:1] + a4t[2:3],
        0.5 * a4t[1:2] + a4t[3:4],
    ], axis=0)
    c_ref[...] = beff + 0.5 * a4t[0:1] + 0.5 * a4t[1:2] + a4t[2:3] + a4t[3:4]


def _main_kernel(boxes_ref, dist_ref, feat_ref, b5_ref, c_ref,
                 wdt_ref, w1fp_ref, b1_ref,
                 z_ref, s_ref, ss_ref):
    i = pl.program_id(0)
    pe = jnp.maximum(
        jnp.dot(boxes_ref[...], b5_ref[...], preferred_element_type=jnp.float32)
        + c_ref[...], 0.0)
    x = jnp.concatenate([feat_ref[...].astype(jnp.bfloat16),
                         pe.astype(jnp.bfloat16)], axis=1)
    z = (jax.lax.dot_general(x, w1fp_ref[...], _NT,
                             preferred_element_type=jnp.float32)
         + jnp.dot(dist_ref[...].astype(jnp.bfloat16), wdt_ref[...],
                   preferred_element_type=jnp.float32)
         + b1_ref[...])
    z_ref[...] = z.astype(jnp.bfloat16)
    zs = jnp.sum(z, axis=0, keepdims=True)
    zss = jnp.sum(z * z, axis=0, keepdims=True)

    @pl.when(i == 0)
    def _init():
        s_ref[...] = zs
        ss_ref[...] = zss

    @pl.when(i > 0)
    def _acc():
        s_ref[...] += zs
        ss_ref[...] += zss


def _finish_kernel(z_ref, s_ref, ss_ref, g2_ref, be2_ref, w2_ref, b2_ref,
                   out_ref):
    mu = s_ref[...] / N
    var = ss_ref[...] / N - mu * mu
    scale = g2_ref[...] * jax.lax.rsqrt(var + EPS)
    shift = be2_ref[...] - mu * scale
    h = jnp.maximum(z_ref[...].astype(jnp.float32) * scale + shift, 0.0)
    out_ref[...] = (jax.lax.dot_general(h.astype(jnp.bfloat16), w2_ref[...],
                                        _NT,
                                        preferred_element_type=jnp.float32)
                    + b2_ref[...])


def kernel(distribution, boxes, features, labels, obj_embed_w, bn4_gamma,
           bn4_beta, pos_W, pos_b, W1, b1, bn2_gamma, bn2_beta, W2, b2):
    b5, c = pl.pallas_call(
        _stats_kernel,
        out_shape=(jax.ShapeDtypeStruct((5, 128), jnp.float32),
                   jax.ShapeDtypeStruct((1, 128), jnp.float32)),
    )(boxes.reshape(625, 160), pos_W, pos_b[None, :],
      bn4_gamma[None, :], bn4_beta[None, :])

    w1fp = jnp.concatenate([W1[:, :2048], W1[:, 2248:2376]],
                           axis=1).astype(jnp.bfloat16)
    wdt = (obj_embed_w @ W1[:, 2048:2248].T).astype(jnp.bfloat16)
    w2b = W2.astype(jnp.bfloat16)
    b1r = b1[None, :]

    z, s, ss = pl.pallas_call(
        _main_kernel,
        grid=(NB,),
        in_specs=[
            pl.BlockSpec((R, 5), lambda i: (i, 0)),
            pl.BlockSpec((R, 36), lambda i: (i, 0)),
            pl.BlockSpec((R, 2048), lambda i: (i, 0)),
            pl.BlockSpec((5, 128), lambda i: (0, 0)),
            pl.BlockSpec((1, 128), lambda i: (0, 0)),
            pl.BlockSpec((36, 1024), lambda i: (0, 0)),
            pl.BlockSpec((1024, 2176), lambda i: (0, 0)),
            pl.BlockSpec((1, 1024), lambda i: (0, 0)),
        ],
        out_specs=(
            pl.BlockSpec((R, 1024), lambda i: (i, 0)),
            pl.BlockSpec((1, 1024), lambda i: (0, 0)),
            pl.BlockSpec((1, 1024), lambda i: (0, 0)),
        ),
        out_shape=(
            jax.ShapeDtypeStruct((N, 1024), jnp.bfloat16),
            jax.ShapeDtypeStruct((1, 1024), jnp.float32),
            jax.ShapeDtypeStruct((1, 1024), jnp.float32),
        ),
        compiler_params=pltpu.CompilerParams(
            dimension_semantics=("arbitrary",)),
    )(boxes, distribution, features, b5, c, wdt, w1fp, b1r)

    dist_out = pl.pallas_call(
        _finish_kernel,
        grid=(NB,),
        in_specs=[
            pl.BlockSpec((R, 1024), lambda i: (i, 0)),
            pl.BlockSpec((1, 1024), lambda i: (0, 0)),
            pl.BlockSpec((1, 1024), lambda i: (0, 0)),
            pl.BlockSpec((1, 1024), lambda i: (0, 0)),
            pl.BlockSpec((1, 1024), lambda i: (0, 0)),
            pl.BlockSpec((37, 1024), lambda i: (0, 0)),
            pl.BlockSpec((1, 37), lambda i: (0, 0)),
        ],
        out_specs=pl.BlockSpec((R, 37), lambda i: (i, 0)),
        out_shape=jax.ShapeDtypeStruct((N, 37), jnp.float32),
        compiler_params=pltpu.CompilerParams(
            dimension_semantics=("arbitrary",)),
    )(z, s, ss, bn2_gamma[None, :], bn2_beta[None, :], w2b, b2[None, :])

    return (dist_out, labels)


# speedup vs baseline: 1.0485x; 1.0485x over previous
import jax
import jax.numpy as jnp
from jax.experimental import pallas as pl
from jax.experimental.pallas import tpu as pltpu

N = 20000
R = 1000
NB = N // R
EPS = 1e-5

_NT = (((1,), (1,)), ((), ()))


def _stats_kernel(bp_ref, posw_ref, posb_ref, g4_ref, be4_ref,
                  b5_ref, c_ref):
    x = bp_ref[...]
    r = jnp.roll(x, -2, axis=1)
    wh = r - x + 1.0
    ctr = x + 0.5 * wh
    rc = jnp.sum(ctr, axis=0, keepdims=True)
    rw = jnp.sum(wh, axis=0, keepdims=True)
    rc2 = jnp.sum(ctr * ctr, axis=0, keepdims=True)
    rw2 = jnp.sum(wh * wh, axis=0, keepdims=True)
    for k in (5, 10, 20, 40, 80):
        rc = rc + jnp.roll(rc, -k, axis=1)
        rw = rw + jnp.roll(rw, -k, axis=1)
        rc2 = rc2 + jnp.roll(rc2, -k, axis=1)
        rw2 = rw2 + jnp.roll(rw2, -k, axis=1)
    mu4 = jnp.concatenate([rc[:, 1:3], rw[:, 1:3]], axis=1) / N
    ex2 = jnp.concatenate([rc2[:, 1:3], rw2[:, 1:3]], axis=1) / N
    var4 = ex2 - mu4 * mu4
    scale4 = g4_ref[...] * jax.lax.rsqrt(var4 + EPS)
    shift4 = be4_ref[...] - mu4 * scale4
    a4t = (posw_ref[...] * scale4).T
    beff = posb_ref[...] + jnp.sum(posw_ref[...] * shift4, axis=1)[None, :]
    zrow = jnp.zeros((1, 128), jnp.float32)
    b5_ref[...] = jnp.concatenate([
        zrow,
        0.5 * a4t[0:1] - a4t[2:3],
        0.5 * a4t[1:2] - a4t[3:4],
        0.5 * a4t[0:1] + a4t[2:3],
        0.5 * a4t[1:2] + a4t[3:4],
    ], axis=0)
    c_ref[...] = beff + 0.5 * a4t[0:1] + 0.5 * a4t[1:2] + a4t[2:3] + a4t[3:4]


def _main_kernel(boxes_ref, dist_ref, feat_ref, b5_ref, c_ref,
                 wdt_ref, w1p_ref, w1f_ref, b1_ref,
                 z_ref, s_ref, ss_ref):
    i = pl.program_id(0)
    pe = jnp.maximum(
        jnp.dot(boxes_ref[...], b5_ref[...], preferred_element_type=jnp.float32)
        + c_ref[...], 0.0)
    z = (jax.lax.dot_general(feat_ref[...].astype(jnp.bfloat16),
                             w1f_ref[...], _NT,
                             preferred_element_type=jnp.float32)
         + jnp.dot(dist_ref[...].astype(jnp.bfloat16), wdt_ref[...],
                   preferred_element_type=jnp.float32)
         + jax.lax.dot_general(pe.astype(jnp.bfloat16), w1p_ref[...],
                               _NT, preferred_element_type=jnp.float32)
         + b1_ref[...])
    z_ref[...] = z.astype(jnp.bfloat16)
    zs = jnp.sum(z, axis=0, keepdims=True)
    zss = jnp.sum(z * z, axis=0, keepdims=True)

    @pl.when(i == 0)
    def _init():
        s_ref[...] = zs
        ss_ref[...] = zss

    @pl.when(i > 0)
    def _acc():
        s_ref[...] += zs
        ss_ref[...] += zss


def _finish_kernel(z_ref, s_ref, ss_ref, g2_ref, be2_ref, w2_ref, b2_ref,
                   out_ref):
    mu = s_ref[...] / N
    var = ss_ref[...] / N - mu * mu
    scale = g2_ref[...] * jax.lax.rsqrt(var + EPS)
    shift = be2_ref[...] - mu * scale
    h = jnp.maximum(z_ref[...].astype(jnp.float32) * scale + shift, 0.0)
    out_ref[...] = (jax.lax.dot_general(h.astype(jnp.bfloat16), w2_ref[...],
                                        _NT,
                                        preferred_element_type=jnp.float32)
                    + b2_ref[...])


def kernel(distribution, boxes, features, labels, obj_embed_w, bn4_gamma,
           bn4_beta, pos_W, pos_b, W1, b1, bn2_gamma, bn2_beta, W2, b2):
    b5, c = pl.pallas_call(
        _stats_kernel,
        out_shape=(jax.ShapeDtypeStruct((5, 128), jnp.float32),
                   jax.ShapeDtypeStruct((1, 128), jnp.float32)),
    )(boxes.reshape(625, 160), pos_W, pos_b[None, :],
      bn4_gamma[None, :], bn4_beta[None, :])

    w1f = W1[:, :2048].astype(jnp.bfloat16)
    w1p = W1[:, 2248:2376].astype(jnp.bfloat16)
    wdt = (obj_embed_w @ W1[:, 2048:2248].T).astype(jnp.bfloat16)
    w2b = W2.astype(jnp.bfloat16)
    b1r = b1[None, :]

    z, s, ss = pl.pallas_call(
        _main_kernel,
        grid=(NB,),
        in_specs=[
            pl.BlockSpec((R, 5), lambda i: (i, 0)),
            pl.BlockSpec((R, 36), lambda i: (i, 0)),
            pl.BlockSpec((R, 2048), lambda i: (i, 0)),
            pl.BlockSpec((5, 128), lambda i: (0, 0)),
            pl.BlockSpec((1, 128), lambda i: (0, 0)),
            pl.BlockSpec((36, 1024), lambda i: (0, 0)),
            pl.BlockSpec((1024, 128), lambda i: (0, 0)),
            pl.BlockSpec((1024, 2048), lambda i: (0, 0)),
            pl.BlockSpec((1, 1024), lambda i: (0, 0)),
        ],
        out_specs=(
            pl.BlockSpec((R, 1024), lambda i: (i, 0)),
            pl.BlockSpec((1, 1024), lambda i: (0, 0)),
            pl.BlockSpec((1, 1024), lambda i: (0, 0)),
        ),
        out_shape=(
            jax.ShapeDtypeStruct((N, 1024), jnp.bfloat16),
            jax.ShapeDtypeStruct((1, 1024), jnp.float32),
            jax.ShapeDtypeStruct((1, 1024), jnp.float32),
        ),
        compiler_params=pltpu.CompilerParams(
            dimension_semantics=("arbitrary",)),
    )(boxes, distribution, features, b5, c, wdt, w1p, w1f, b1r)

    R2 = 2000
    dist_out = pl.pallas_call(
        _finish_kernel,
        grid=(N // R2,),
        in_specs=[
            pl.BlockSpec((R2, 1024), lambda i: (i, 0)),
            pl.BlockSpec((1, 1024), lambda i: (0, 0)),
            pl.BlockSpec((1, 1024), lambda i: (0, 0)),
            pl.BlockSpec((1, 1024), lambda i: (0, 0)),
            pl.BlockSpec((1, 1024), lambda i: (0, 0)),
            pl.BlockSpec((37, 1024), lambda i: (0, 0)),
            pl.BlockSpec((1, 37), lambda i: (0, 0)),
        ],
        out_specs=pl.BlockSpec((R2, 37), lambda i: (i, 0)),
        out_shape=jax.ShapeDtypeStruct((N, 37), jnp.float32),
        compiler_params=pltpu.CompilerParams(
            dimension_semantics=("parallel",)),
    )(z, s, ss, bn2_gamma[None, :], bn2_beta[None, :], w2b, b2[None, :])

    return (dist_out, labels)
